# baseline (speedup 1.0000x reference)
import functools

import jax
import jax.numpy as jnp
from jax import lax
from jax.experimental import pallas as pl
from jax.experimental.pallas import tpu as pltpu
from jax.experimental.pallas import tpu_sc as plsc

_LANES = 16


def _retile_table(wt, *, num_cores, num_subcores):
    d, v = wt.shape
    nw = num_cores * num_subcores
    cc = 2000
    nch = v // cc

    mesh = plsc.VectorSubcoreMesh(core_axis_name="c", subcore_axis_name="s")

    @functools.partial(
        pl.kernel,
        mesh=mesh,
        out_type=jax.ShapeDtypeStruct((v, d), jnp.float32),
        scratch_types=[
            pltpu.VMEM((d, cc), jnp.float32),
            pltpu.VMEM((cc, d), jnp.float32),
        ],
        compiler_params=pltpu.CompilerParams(
            use_tc_tiling_on_sc=False, needs_layout_passes=False
        ),
    )
    def k(wt_hbm, out_hbm, buf_in, buf_out):
        wid = lax.axis_index("s") * num_cores + lax.axis_index("c")
        n_mine = jnp.where(wid < (nch % nw), nch // nw + 1, nch // nw)

        def chunk_body(i, carry):
            c0 = (wid + i * nw) * cc
            pltpu.sync_copy(wt_hbm.at[:, pl.ds(c0, cc)], buf_in)

            def col_group(g, c2):
                for kk in range(_LANES):
                    col = g * _LANES + kk
                    vals = plsc.load_gather(
                        buf_in,
                        [lax.iota(jnp.int32, _LANES),
                         jnp.full((_LANES,), col, jnp.int32)],
                    )
                    plsc.store_scatter(
                        buf_out,
                        [jnp.full((_LANES,), col, jnp.int32),
                         lax.iota(jnp.int32, _LANES)],
                        vals,
                    )
                return c2

            lax.fori_loop(0, cc // _LANES, col_group, 0)
            pltpu.sync_copy(buf_out, out_hbm.at[pl.ds(c0, cc)])
            return carry

        lax.fori_loop(0, n_mine, chunk_body, 0)

    return k(wt)


def _embedding_planes(xt, weight, *, num_cores, num_subcores):
    f, b = xt.shape
    v, d = weight.shape
    nw = num_cores * num_subcores
    nb = b // nw

    mesh = plsc.VectorSubcoreMesh(core_axis_name="c", subcore_axis_name="s")

    @functools.partial(
        pl.kernel,
        mesh=mesh,
        out_type=jax.ShapeDtypeStruct((f, d, b), jnp.float32),
        scratch_types=[
            pltpu.VMEM((f, nb), jnp.int32),
            pltpu.VMEM((nb, d), jnp.float32),
            pltpu.VMEM((nb, d), jnp.float32),
            pltpu.VMEM((d, nb), jnp.float32),
            pltpu.VMEM((d, nb), jnp.float32),
            pltpu.SemaphoreType.DMA,
            pltpu.SemaphoreType.DMA,
            pltpu.SemaphoreType.DMA,
            pltpu.SemaphoreType.DMA,
        ],
        compiler_params=pltpu.CompilerParams(
            use_tc_tiling_on_sc=False, needs_layout_passes=False
        ),
    )
    def k(xt_hbm, table_hbm, out_hbm,
          idx_v, rows0, rows1, tb0, tb1, g0, g1, s0, s1):
        wid = lax.axis_index("s") * num_cores + lax.axis_index("c")
        base = wid * nb
        pltpu.sync_copy(xt_hbm.at[:, pl.ds(base, nb)], idx_v)

        rows = (rows0, rows1)
        tbs = (tb0, tb1)
        gsems = (g0, g1)
        ssems = (s0, s1)

        def gather(fi):
            return pltpu.async_copy(
                table_hbm.at[idx_v.at[fi]], rows[fi % 2], gsems[fi % 2]
            )

        pending = gather(0)
        stores = [None, None]
        for fi in range(f):
            nxt = gather(fi + 1) if fi + 1 < f else None
            pending.wait()
            if stores[fi % 2] is not None:
                stores[fi % 2].wait()
            r = rows[fi % 2]
            t = tbs[fi % 2]

            def transpose_block(g, carry):
                rid = g * _LANES + lax.iota(jnp.int32, _LANES)
                for h in range(d):
                    col = jnp.full((_LANES,), h, jnp.int32)
                    t[h, pl.ds(g * _LANES, _LANES)] = plsc.load_gather(
                        r, [rid, col]
                    )
                return carry

            lax.fori_loop(0, nb // _LANES, transpose_block, 0)
            stores[fi % 2] = pltpu.async_copy(
                t, out_hbm.at[fi, :, pl.ds(base, nb)], ssems[fi % 2]
            )
            pending = nxt
        for st in stores:
            if st is not None:
                st.wait()

    return k(xt, weight)


def kernel(x, weight):
    b, f = x.shape
    v, d = weight.shape
    xt = x.T.astype(jnp.int32)
    wrow = _retile_table(weight.T, num_cores=2, num_subcores=16)
    out_planes = _embedding_planes(xt, wrow, num_cores=2, num_subcores=16)
    return jnp.transpose(out_planes, (2, 0, 1))


# speedup vs baseline: 2.8866x; 2.8866x over previous
import functools

import jax
import jax.numpy as jnp
from jax import lax
from jax.experimental import pallas as pl
from jax.experimental.pallas import tpu as pltpu
from jax.experimental.pallas import tpu_sc as plsc

_LANES = 16


def _embedding_planes(xt, weight, *, num_cores, num_subcores):
    f, b = xt.shape
    v, d = weight.shape
    nw = num_cores * num_subcores
    nb = b // nw

    mesh = plsc.VectorSubcoreMesh(core_axis_name="c", subcore_axis_name="s")

    @functools.partial(
        pl.kernel,
        mesh=mesh,
        out_type=jax.ShapeDtypeStruct((f, d, b), jnp.float32),
        scratch_types=[
            pltpu.VMEM((f, nb), jnp.int32),
            pltpu.VMEM((nb, d), jnp.float32),
            pltpu.VMEM((nb, d), jnp.float32),
            pltpu.VMEM((nb, d), jnp.float32),
            pltpu.VMEM((d, nb), jnp.float32),
            pltpu.VMEM((d, nb), jnp.float32),
            pltpu.SemaphoreType.DMA,
            pltpu.SemaphoreType.DMA,
            pltpu.SemaphoreType.DMA,
            pltpu.SemaphoreType.DMA,
            pltpu.SemaphoreType.DMA,
        ],
        compiler_params=pltpu.CompilerParams(
            use_tc_tiling_on_sc=False, needs_layout_passes=False
        ),
    )
    def k(xt_hbm, table_hbm, out_hbm,
          idx_v, rows0, rows1, rows2, tb0, tb1, g0, g1, g2, s0, s1):
        wid = lax.axis_index("s") * num_cores + lax.axis_index("c")
        base = wid * nb
        pltpu.sync_copy(xt_hbm.at[:, pl.ds(base, nb)], idx_v)

        rows = (rows0, rows1, rows2)
        tbs = (tb0, tb1)
        gsems = (g0, g1, g2)
        ssems = (s0, s1)

        def gather(fi):
            return pltpu.async_copy(
                table_hbm.at[idx_v.at[fi]], rows[fi % 3], gsems[fi % 3]
            )

        gathers = [gather(0), gather(1)]
        stores = [None, None]
        for fi in range(f):
            if fi + 2 < f:
                gathers.append(gather(fi + 2))
            gathers[fi].wait()
            if stores[fi % 2] is not None:
                stores[fi % 2].wait()
            r = rows[fi % 3]
            t = tbs[fi % 2]

            def transpose_block(g, carry):
                rid = g * _LANES + lax.iota(jnp.int32, _LANES)
                for h in range(d):
                    col = jnp.full((_LANES,), h, jnp.int32)
                    t[h, pl.ds(g * _LANES, _LANES)] = plsc.load_gather(
                        r, [rid, col]
                    )
                return carry

            lax.fori_loop(0, nb // _LANES, transpose_block, 0)
            stores[fi % 2] = pltpu.async_copy(
                t, out_hbm.at[fi, :, pl.ds(base, nb)], ssems[fi % 2]
            )
        for st in stores:
            if st is not None:
                st.wait()

    return k(xt, weight)


def kernel(x, weight):
    b, f = x.shape
    xt = x.T.astype(jnp.int32)
    out_planes = _embedding_planes(xt, weight, num_cores=2, num_subcores=16)
    return jnp.transpose(out_planes, (2, 0, 1))
